# Initial kernel scaffold; baseline (speedup 1.0000x reference)
import jax
import jax.numpy as jnp
from jax.experimental import pallas as pl


def kernel(x, edge_index, W_est, b_est, fc_est_W, fc_est_b, W_gnn, b_gnn, cls_W, cls_b):
    raise NotImplementedError("write your pallas kernel here")


# speedup vs baseline: 17.5174x; 17.5174x over previous
import functools

import jax
import jax.numpy as jnp
from jax import lax
from jax.experimental import pallas as pl
from jax.experimental.pallas import tpu as pltpu
from jax.experimental.pallas import tpu_sc as plsc

N = 10000
E = 320000
D = 128
HE = 64
H = 128

NC = 2
NS = 16
NW = NC * NS
CH = 128
NCHUNK = 79
EPT = NCHUNK * CH
EPAD = NW * EPT
NP = 10240
W = 8

_mesh = plsc.VectorSubcoreMesh(core_axis_name="c", subcore_axis_name="s")
_sc_params = pltpu.CompilerParams(use_tc_tiling_on_sc=False)


@functools.partial(
    pl.kernel,
    out_type=jax.ShapeDtypeStruct((NC, NP, W), jnp.float32),
    mesh=_mesh,
    scratch_types=[
        pltpu.VMEM((CH,), jnp.int32),
        pltpu.VMEM((CH,), jnp.int32),
        pltpu.VMEM((CH, W), jnp.float32),
        pltpu.VMEM((CH, W), jnp.float32),
        pltpu.VMEM_SHARED((NP, W), jnp.float32),
    ],
    compiler_params=_sc_params,
)
def _k1_degrees(edge_hbm, onesa_hbm, onesb_hbm, zeros_hbm, degp_hbm,
                src_v, dst_v, onesa_v, onesb_v, deg_s):
    cid = lax.axis_index("c")
    sid = lax.axis_index("s")
    wid = sid * NC + cid
    pltpu.sync_copy(onesa_hbm, onesa_v)
    pltpu.sync_copy(onesb_hbm, onesb_v)

    @pl.when(sid == 0)
    def _():
        pltpu.sync_copy(zeros_hbm, deg_s)

    plsc.subcore_barrier()

    def body(j, c):
        pltpu.sync_copy(edge_hbm.at[0, wid, j], src_v)
        pltpu.sync_copy(edge_hbm.at[1, wid, j], dst_v)
        pltpu.sync_copy(onesa_v, deg_s.at[src_v], add=True)
        pltpu.sync_copy(onesb_v, deg_s.at[dst_v], add=True)
        return c

    lax.fori_loop(0, NCHUNK, body, 0)
    plsc.subcore_barrier()

    @pl.when(sid == 0)
    def _():
        pltpu.sync_copy(deg_s, degp_hbm.at[cid])


def _k2_body(degp_ref, x_ref, we_ref, fe_ref, wg_ref, cw_ref,
             norms_ref, v_ref):
    deg = degp_ref[0] + degp_ref[1]
    norms = lax.rsqrt(jnp.maximum(deg[:, 0:2], 1.0))
    norms_ref[...] = norms
    wv0 = jnp.dot(we_ref[...], fe_ref[...], preferred_element_type=jnp.float32)
    wv1 = jnp.dot(wg_ref[...], cw_ref[...], preferred_element_type=jnp.float32)
    wv = jnp.concatenate([wv0, wv1], axis=1)
    u = jnp.dot(x_ref[...], wv, preferred_element_type=jnp.float32)
    v2 = u * norms[:, 0:1]
    v_ref[...] = jnp.concatenate(
        [v2, jnp.zeros((NP, W - 2), jnp.float32)], axis=1)


_k2 = pl.pallas_call(
    _k2_body,
    out_shape=(
        jax.ShapeDtypeStruct((NP, 2), jnp.float32),
        jax.ShapeDtypeStruct((NP, W), jnp.float32),
    ),
)


@functools.partial(
    pl.kernel,
    out_type=jax.ShapeDtypeStruct((NC, NP, W), jnp.float32),
    mesh=_mesh,
    scratch_types=[
        pltpu.VMEM((CH,), jnp.int32),
        pltpu.VMEM((CH,), jnp.int32),
        pltpu.VMEM((CH, W), jnp.float32),
        pltpu.VMEM_SHARED((NP, W), jnp.float32),
        pltpu.SemaphoreType.DMA,
    ],
    compiler_params=_sc_params,
)
def _k3_edges(edge_hbm, v_hbm, zeros_hbm, aggp_hbm,
              src_v, dst_v, vals_v, agg_s, gsem):
    cid = lax.axis_index("c")
    sid = lax.axis_index("s")
    wid = sid * NC + cid

    @pl.when(sid == 0)
    def _():
        pltpu.sync_copy(zeros_hbm, agg_s)

    plsc.subcore_barrier()

    def body(j, c):
        pltpu.sync_copy(edge_hbm.at[0, wid, j], src_v)
        pltpu.sync_copy(edge_hbm.at[1, wid, j], dst_v)
        pltpu.async_copy(v_hbm.at[src_v], vals_v, gsem).wait()
        pltpu.sync_copy(vals_v, agg_s.at[dst_v], add=True)
        return c

    lax.fori_loop(0, NCHUNK, body, 0)
    plsc.subcore_barrier()

    @pl.when(sid == 0)
    def _():
        pltpu.sync_copy(agg_s, aggp_hbm.at[cid])


def _k4_body(aggp_ref, ni_ref, be_ref, fe_ref, feb_ref, bg_ref, cw_ref,
             cb_ref, y_ref, s_ref):
    agg = aggp_ref[0] + aggp_ref[1]
    a = agg[:N]
    ni = ni_ref[...]
    cs = jnp.sum(jnp.dot(be_ref[...], fe_ref[...],
                         preferred_element_type=jnp.float32)) + jnp.sum(feb_ref[...])
    cy = jnp.sum(jnp.dot(bg_ref[...], cw_ref[...],
                         preferred_element_type=jnp.float32)) + jnp.sum(cb_ref[...])
    s_ref[...] = a[:, 0:1] * ni + cs
    y_ref[...] = a[:, 1:2] * ni + cy


_k4 = pl.pallas_call(
    _k4_body,
    out_shape=(
        jax.ShapeDtypeStruct((N, 1), jnp.float32),
        jax.ShapeDtypeStruct((N, 1), jnp.float32),
    ),
)


def kernel(x, edge_index, W_est, b_est, fc_est_W, fc_est_b, W_gnn, b_gnn,
           cls_W, cls_b):
    pad_e = jnp.full((2, EPAD - E), N, jnp.int32)
    edge_r = jnp.concatenate([edge_index, pad_e], axis=1).reshape(2, NW, NCHUNK, CH)
    eyeW = jnp.eye(W, dtype=jnp.float32)
    onesa = jnp.tile(eyeW[0:1], (CH, 1))
    onesb = jnp.tile(eyeW[1:2], (CH, 1))
    zeros = jnp.zeros((NP, W), jnp.float32)
    x_pad = jnp.pad(x, ((0, NP - N), (0, 0)))

    degp = _k1_degrees(edge_r, onesa, onesb, zeros)
    norms, v = _k2(degp, x_pad, W_est, fc_est_W, W_gnn, cls_W)
    aggp = _k3_edges(edge_r, v, zeros)
    ni_col = norms[:N, 1:2]
    y, s = _k4(aggp, ni_col, b_est.reshape(1, HE), fc_est_W,
               fc_est_b.reshape(1, 1), b_gnn.reshape(1, H), cls_W,
               cls_b.reshape(1, 1))
    return (y, s)


# speedup vs baseline: 41.4285x; 2.3650x over previous
import functools

import jax
import jax.numpy as jnp
from jax import lax
from jax.experimental import pallas as pl
from jax.experimental.pallas import tpu as pltpu
from jax.experimental.pallas import tpu_sc as plsc

N = 10000
E = 320000
D = 128
HE = 64
H = 128

NC = 2
NS = 16
NW = NC * NS
CH = 128
NCHUNK = 79
EPT = NCHUNK * CH
EPAD = NW * EPT
NP = 10240
W = 8

_mesh = plsc.VectorSubcoreMesh(core_axis_name="c", subcore_axis_name="s")
_sc_params = pltpu.CompilerParams(use_tc_tiling_on_sc=False)


@functools.partial(
    pl.kernel,
    out_type=jax.ShapeDtypeStruct((NC, NP, W), jnp.float32),
    mesh=_mesh,
    scratch_types=[
        pltpu.VMEM((NCHUNK, CH), jnp.int32),
        pltpu.VMEM((NCHUNK, CH), jnp.int32),
        pltpu.VMEM((CH, W), jnp.float32),
        pltpu.VMEM((CH, W), jnp.float32),
        pltpu.VMEM_SHARED((NP, W), jnp.float32),
        pltpu.SemaphoreType.DMA,
    ],
    compiler_params=_sc_params,
)
def _k1_degrees(edge_hbm, onesa_hbm, onesb_hbm, zeros_hbm, degp_hbm,
                src_v, dst_v, onesa_v, onesb_v, deg_s, sem):
    cid = lax.axis_index("c")
    sid = lax.axis_index("s")
    wid = sid * NC + cid
    pltpu.sync_copy(onesa_hbm, onesa_v)
    pltpu.sync_copy(onesb_hbm, onesb_v)
    pltpu.sync_copy(edge_hbm.at[0, wid], src_v)
    pltpu.sync_copy(edge_hbm.at[1, wid], dst_v)

    @pl.when(sid == 0)
    def _():
        pltpu.sync_copy(zeros_hbm, deg_s)

    plsc.subcore_barrier()

    def fire(j, c):
        pltpu.async_copy(onesa_v, deg_s.at[src_v.at[j]], sem, add=True)
        pltpu.async_copy(onesb_v, deg_s.at[dst_v.at[j]], sem, add=True)
        return c

    lax.fori_loop(0, NCHUNK, fire, 0)

    def drain(j, c):
        pltpu.make_async_copy(onesa_v, deg_s.at[src_v.at[j]], sem).wait()
        pltpu.make_async_copy(onesb_v, deg_s.at[dst_v.at[j]], sem).wait()
        return c

    lax.fori_loop(0, NCHUNK, drain, 0)
    plsc.subcore_barrier()

    @pl.when(sid == 0)
    def _():
        pltpu.sync_copy(deg_s, degp_hbm.at[cid])


def _k2_body(degp_ref, x_ref, we_ref, fe_ref, wg_ref, cw_ref,
             norms_ref, v_ref):
    deg = degp_ref[0] + degp_ref[1]
    norms = lax.rsqrt(jnp.maximum(deg[:, 0:2], 1.0))
    norms_ref[...] = norms
    wv0 = jnp.dot(we_ref[...], fe_ref[...], preferred_element_type=jnp.float32)
    wv1 = jnp.dot(wg_ref[...], cw_ref[...], preferred_element_type=jnp.float32)
    wv = jnp.concatenate([wv0, wv1], axis=1)
    u = jnp.dot(x_ref[...], wv, preferred_element_type=jnp.float32)
    v2 = u * norms[:, 0:1]
    v_ref[...] = jnp.concatenate(
        [v2, jnp.zeros((NP, W - 2), jnp.float32)], axis=1)


_k2 = pl.pallas_call(
    _k2_body,
    out_shape=(
        jax.ShapeDtypeStruct((NP, 2), jnp.float32),
        jax.ShapeDtypeStruct((NP, W), jnp.float32),
    ),
)


@functools.partial(
    pl.kernel,
    out_type=jax.ShapeDtypeStruct((NC, NP, W), jnp.float32),
    mesh=_mesh,
    scratch_types=[
        pltpu.VMEM((NCHUNK, CH), jnp.int32),
        pltpu.VMEM((NCHUNK, CH), jnp.int32),
        pltpu.VMEM((NCHUNK, CH, W), jnp.float32),
        pltpu.VMEM_SHARED((NP, W), jnp.float32),
        pltpu.SemaphoreType.DMA,
        pltpu.SemaphoreType.DMA,
    ],
    compiler_params=_sc_params,
)
def _k3_edges(edge_hbm, v_hbm, zeros_hbm, aggp_hbm,
              src_v, dst_v, vals_v, agg_s, gsem, ssem):
    cid = lax.axis_index("c")
    sid = lax.axis_index("s")
    wid = sid * NC + cid
    pltpu.sync_copy(edge_hbm.at[0, wid], src_v)
    pltpu.sync_copy(edge_hbm.at[1, wid], dst_v)

    @pl.when(sid == 0)
    def _():
        pltpu.sync_copy(zeros_hbm, agg_s)

    def gfire(j, c):
        pltpu.async_copy(v_hbm.at[src_v.at[j]], vals_v.at[j], gsem)
        return c

    lax.fori_loop(0, NCHUNK, gfire, 0)

    def gdrain(j, c):
        pltpu.make_async_copy(v_hbm.at[src_v.at[j]], vals_v.at[j], gsem).wait()
        return c

    lax.fori_loop(0, NCHUNK, gdrain, 0)
    plsc.subcore_barrier()

    def sfire(j, c):
        pltpu.async_copy(vals_v.at[j], agg_s.at[dst_v.at[j]], ssem, add=True)
        return c

    lax.fori_loop(0, NCHUNK, sfire, 0)

    def sdrain(j, c):
        pltpu.make_async_copy(vals_v.at[j], agg_s.at[dst_v.at[j]], ssem).wait()
        return c

    lax.fori_loop(0, NCHUNK, sdrain, 0)
    plsc.subcore_barrier()

    @pl.when(sid == 0)
    def _():
        pltpu.sync_copy(agg_s, aggp_hbm.at[cid])


def _k4_body(aggp_ref, ni_ref, be_ref, fe_ref, feb_ref, bg_ref, cw_ref,
             cb_ref, y_ref, s_ref):
    agg = aggp_ref[0] + aggp_ref[1]
    a = agg[:N]
    ni = ni_ref[...]
    cs = jnp.sum(jnp.dot(be_ref[...], fe_ref[...],
                         preferred_element_type=jnp.float32)) + jnp.sum(feb_ref[...])
    cy = jnp.sum(jnp.dot(bg_ref[...], cw_ref[...],
                         preferred_element_type=jnp.float32)) + jnp.sum(cb_ref[...])
    s_ref[...] = a[:, 0:1] * ni + cs
    y_ref[...] = a[:, 1:2] * ni + cy


_k4 = pl.pallas_call(
    _k4_body,
    out_shape=(
        jax.ShapeDtypeStruct((N, 1), jnp.float32),
        jax.ShapeDtypeStruct((N, 1), jnp.float32),
    ),
)


def kernel(x, edge_index, W_est, b_est, fc_est_W, fc_est_b, W_gnn, b_gnn,
           cls_W, cls_b):
    pad_e = jnp.full((2, EPAD - E), N, jnp.int32)
    edge_r = jnp.concatenate([edge_index, pad_e], axis=1).reshape(2, NW, NCHUNK, CH)
    eyeW = jnp.eye(W, dtype=jnp.float32)
    onesa = jnp.tile(eyeW[0:1], (CH, 1))
    onesb = jnp.tile(eyeW[1:2], (CH, 1))
    zeros = jnp.zeros((NP, W), jnp.float32)
    x_pad = jnp.pad(x, ((0, NP - N), (0, 0)))

    degp = _k1_degrees(edge_r, onesa, onesb, zeros)
    norms, v = _k2(degp, x_pad, W_est, fc_est_W, W_gnn, cls_W)
    aggp = _k3_edges(edge_r, v, zeros)
    ni_col = norms[:N, 1:2]
    y, s = _k4(aggp, ni_col, b_est.reshape(1, HE), fc_est_W,
               fc_est_b.reshape(1, 1), b_gnn.reshape(1, H), cls_W,
               cls_b.reshape(1, 1))
    return (y, s)
